# baseline (device time: 98029 ns/iter reference)
import jax
import jax.numpy as jnp
from jax import lax
from jax.experimental import pallas as pl
from jax.experimental.pallas import tpu as pltpu

Y = 4
C = 4


def kernel(x):
    m, n = x.shape
    n_out = n // Y
    mc = m // C

    def body(x_ref, out_ref, stage_ref, xbf_ref, load_sems, send_sems, recv_sems):
        mx = lax.axis_index("x")
        my = lax.axis_index("y")
        mz = lax.axis_index("z")

        barrier = pltpu.get_barrier_semaphore()
        for d in range(1, Y):
            k = (my + d) % Y
            pl.semaphore_signal(
                barrier, inc=1,
                device_id=(mx, k, mz), device_id_type=pl.DeviceIdType.MESH,
            )

        def start_load(c):
            cp = pltpu.make_async_copy(
                x_ref.at[pl.ds(c * mc, mc), :],
                stage_ref.at[c % 2],
                load_sems.at[c % 2],
            )
            cp.start()
            return cp

        pending = start_load(0)
        rdmas = []
        for c in range(C):
            pending.wait()
            if c + 1 < C:
                pending = start_load(c + 1)
            xbf_ref[pl.ds(c * mc, mc), :] = stage_ref[c % 2].astype(jnp.bfloat16)
            if c == 0:
                pl.semaphore_wait(barrier, Y - 1)
            for d in range(1, Y):
                k = (my + d) % Y
                rdma = pltpu.make_async_remote_copy(
                    src_ref=xbf_ref.at[pl.ds(c * mc, mc), pl.ds(k * n_out, n_out)],
                    dst_ref=out_ref.at[pl.ds(my * m + c * mc, mc), :],
                    send_sem=send_sems.at[d - 1, c],
                    recv_sem=recv_sems.at[d - 1, c],
                    device_id=(mx, k, mz),
                    device_id_type=pl.DeviceIdType.MESH,
                )
                rdma.start()
                rdmas.append(rdma)
            out_ref[pl.ds(my * m + c * mc, mc), :] = (
                xbf_ref[pl.ds(c * mc, mc), pl.ds(my * n_out, n_out)]
            )

        for rdma in rdmas:
            rdma.wait_send()
            rdma.wait_recv()

    return pl.pallas_call(
        body,
        out_shape=jax.ShapeDtypeStruct((Y * m, n_out), jnp.bfloat16),
        in_specs=[pl.BlockSpec(memory_space=pl.ANY)],
        out_specs=pl.BlockSpec(memory_space=pltpu.VMEM),
        scratch_shapes=[
            pltpu.VMEM((2, mc, n), x.dtype),
            pltpu.VMEM((m, n), jnp.bfloat16),
            pltpu.SemaphoreType.DMA((2,)),
            pltpu.SemaphoreType.DMA((Y - 1, C)),
            pltpu.SemaphoreType.DMA((Y - 1, C)),
        ],
        compiler_params=pltpu.CompilerParams(collective_id=0),
    )(x)
